# baseline (speedup 1.0000x reference)
import functools

import jax
import jax.numpy as jnp
from jax import lax
from jax.experimental import pallas as pl
from jax.experimental.pallas import tpu as pltpu
from jax.experimental.pallas import tpu_sc as plsc

NC = 2
NS = 16
NW = NC * NS
LANES = 16

B, L, K, D = 4096, 50, 4, 32
BW = B // NW
NIDX = K * BW

VB = 65536
VC = 512
V = 1000000
NBLK = -(-V // VB)
VPAD = NBLK * VB


def _tc_relayout_body(t_ref, out_ref):
    for c in range(VB // VC):
        x4 = jnp.concatenate(
            [t_ref[:, c * VC + a * 128:c * VC + (a + 1) * 128]
             for a in range(4)], axis=0)
        out_ref[c * (VC // 4):(c + 1) * (VC // 4), :] = jnp.transpose(x4)


@functools.lru_cache(maxsize=None)
def _tc_relayout():
    return pl.pallas_call(
        _tc_relayout_body,
        grid=(NBLK,),
        in_specs=[pl.BlockSpec((D, VB), lambda i: (0, i))],
        out_specs=pl.BlockSpec((VB // 4, 128), lambda i: (i, 0)),
        out_shape=jax.ShapeDtypeStruct((VPAD // 4, 128), jnp.float32),
    )


@functools.lru_cache(maxsize=None)
def _sc_gather():
    mesh = plsc.VectorSubcoreMesh(core_axis_name="c", subcore_axis_name="s")

    @functools.partial(
        pl.kernel,
        mesh=mesh,
        compiler_params=pltpu.CompilerParams(use_tc_tiling_on_sc=False),
        out_type=jax.ShapeDtypeStruct((L, NW, BW, D), jnp.float32),
        scratch_types=[
            pltpu.VMEM((NIDX,), jnp.int32),
            pltpu.VMEM((NIDX,), jnp.int32),
            pltpu.VMEM((NIDX,), jnp.int32),
            pltpu.VMEM((NIDX,), jnp.int32),
            pltpu.VMEM((NIDX, D), jnp.float32),
            pltpu.VMEM((NIDX, D), jnp.float32),
            pltpu.VMEM((BW, D), jnp.float32),
            pltpu.VMEM((BW, D), jnp.float32),
            pltpu.SemaphoreType.DMA,
            pltpu.SemaphoreType.DMA,
            pltpu.SemaphoreType.DMA,
        ],
    )
    def k(idx_hbm, table_hbm, out_hbm, x0, x1, p0, p1, r0, r1, o0, o1,
          isem, gsem, osem):
        wid = lax.axis_index("s") * NC + lax.axis_index("c")
        xs, ps, rs, os_ = (x0, x1), (p0, p1), (r0, r1), (o0, o1)

        def idx_dma(l):
            return pltpu.make_async_copy(
                idx_hbm.at[l, wid], xs[l % 2], isem)

        def gather_dma(l):
            return pltpu.make_async_copy(
                table_hbm.at[ps[l % 2]], rs[l % 2], gsem)

        def out_dma(l):
            return pltpu.make_async_copy(
                os_[l % 2], out_hbm.at[l, wid], osem)

        def remap(l):
            x, p = xs[l % 2], ps[l % 2]

            def body(i, _):
                sl = pl.ds(i * LANES, LANES)
                v = x[sl]
                p[sl] = ((v & -512) + ((v & 127) << 2)
                         + ((v >> 7) & 3))
                return 0

            lax.fori_loop(0, NIDX // LANES, body, 0)

        def compute(l):
            r, o = rs[l % 2], os_[l % 2]

            UNROLL = 4

            def bo_body(i, _):
                base = i * UNROLL
                for u in range(UNROLL):
                    bo = base + u
                    for half in range(D // LANES):
                        sl = pl.ds(half * LANES, LANES)
                        acc = r[bo, sl]
                        for kk in range(1, K):
                            acc = acc + r[kk * BW + bo, sl]
                        o[bo, sl] = acc
                return 0

            lax.fori_loop(0, BW // UNROLL, bo_body, 0)

        idx_dma(0).start()
        idx_dma(0).wait()
        remap(0)
        gather_dma(0).start()
        idx_dma(1).start()
        pending_out = []
        for l in range(L):
            gather_dma(l).wait()
            if l + 2 < L:
                idx_dma(l + 2).start()
            if l + 1 < L:
                idx_dma(l + 1).wait()
                remap(l + 1)
                gather_dma(l + 1).start()
            if len(pending_out) == 2:
                pending_out.pop(0).wait()
            compute(l)
            dma = out_dma(l)
            dma.start()
            pending_out.append(dma)
        for dma in pending_out:
            dma.wait()

    return k


def kernel(word_inputs, word_seq_lengths, word_embedding_table):
    idx5 = (word_inputs.astype(jnp.int32)
            .reshape(NW, BW, L, K)
            .transpose(2, 0, 3, 1)
            .reshape(L, NW, NIDX))
    tperm = _tc_relayout()(word_embedding_table.T)
    table_rows = tperm.reshape(VPAD, D)
    out = _sc_gather()(idx5, table_rows)
    return out.reshape(L, B, D).transpose(1, 0, 2)


# speedup vs baseline: 1.0847x; 1.0847x over previous
import functools

import jax
import jax.numpy as jnp
from jax import lax
from jax.experimental import pallas as pl
from jax.experimental.pallas import tpu as pltpu
from jax.experimental.pallas import tpu_sc as plsc

NC = 2
NS = 16
NW = NC * NS
LANES = 16

B, L, K, D = 4096, 50, 4, 32
BW = B // NW
NIDX = K * BW

VB = 65536
VC = 512
V = 1000000
NBLK = -(-V // VB)
VPAD = NBLK * VB


def _tc_relayout_body(t_ref, out_ref):
    for c in range(VB // VC):
        x4 = jnp.concatenate(
            [t_ref[:, c * VC + a * 128:c * VC + (a + 1) * 128]
             for a in range(4)], axis=0)
        out_ref[c * (VC // 4):(c + 1) * (VC // 4), :] = jnp.transpose(x4)


@functools.lru_cache(maxsize=None)
def _tc_relayout():
    return pl.pallas_call(
        _tc_relayout_body,
        grid=(NBLK,),
        in_specs=[pl.BlockSpec((D, VB), lambda i: (0, i))],
        out_specs=pl.BlockSpec((VB // 4, 128), lambda i: (i, 0)),
        out_shape=jax.ShapeDtypeStruct((VPAD // 4, 128), jnp.float32),
    )


@functools.lru_cache(maxsize=None)
def _sc_gather():
    mesh = plsc.VectorSubcoreMesh(core_axis_name="c", subcore_axis_name="s")

    @functools.partial(
        pl.kernel,
        mesh=mesh,
        compiler_params=pltpu.CompilerParams(use_tc_tiling_on_sc=False),
        out_type=jax.ShapeDtypeStruct((L, NW, BW, D), jnp.float32),
        scratch_types=[
            pltpu.VMEM((2 * NIDX,), jnp.int32),
            pltpu.VMEM((2 * NIDX,), jnp.int32),
            pltpu.VMEM((2 * NIDX,), jnp.int32),
            pltpu.VMEM((2 * NIDX,), jnp.int32),
            pltpu.VMEM((2 * NIDX, D), jnp.float32),
            pltpu.VMEM((2 * NIDX, D), jnp.float32),
            pltpu.VMEM((2 * BW, D), jnp.float32),
            pltpu.VMEM((2 * BW, D), jnp.float32),
            pltpu.SemaphoreType.DMA,
            pltpu.SemaphoreType.DMA,
            pltpu.SemaphoreType.DMA,
        ],
    )
    def k(idx_hbm, table_hbm, out_hbm, x0, x1, p0, p1, r0, r1, o0, o1,
          isem, gsem, osem):
        wid = lax.axis_index("s") * NC + lax.axis_index("c")
        xs, ps, rs, os_ = (x0, x1), (p0, p1), (r0, r1), (o0, o1)
        NT = L // 2

        def idx_dmas(t):
            x = xs[t % 2]
            return [pltpu.make_async_copy(
                        idx_hbm.at[2 * t + j, wid],
                        x.at[pl.ds(j * NIDX, NIDX)], isem)
                    for j in range(2)]

        def gather_dma(t):
            return pltpu.make_async_copy(
                table_hbm.at[ps[t % 2]], rs[t % 2], gsem)

        def out_dmas(t):
            o = os_[t % 2]
            return [pltpu.make_async_copy(
                        o.at[pl.ds(j * BW, BW)],
                        out_hbm.at[2 * t + j, wid], osem)
                    for j in range(2)]

        def remap(t):
            x, p = xs[t % 2], ps[t % 2]

            def body(i, _):
                sl = pl.ds(i * LANES, LANES)
                v = x[sl]
                p[sl] = ((v & -512) + ((v & 127) << 2)
                         + ((v >> 7) & 3))
                return 0

            lax.fori_loop(0, 2 * NIDX // LANES, body, 0)

        def compute(t):
            r, o = rs[t % 2], os_[t % 2]

            def bo_body(bo, _):
                for j in range(2):
                    for half in range(D // LANES):
                        sl = pl.ds(half * LANES, LANES)
                        acc = r[j * NIDX + bo, sl]
                        for kk in range(1, K):
                            acc = acc + r[j * NIDX + kk * BW + bo, sl]
                        o[j * BW + bo, sl] = acc
                return 0

            lax.fori_loop(0, BW, bo_body, 0)

        for d in idx_dmas(0):
            d.start()
        for d in idx_dmas(0):
            d.wait()
        remap(0)
        gather_dma(0).start()
        for d in idx_dmas(1):
            d.start()
        pending_out = []
        for t in range(NT):
            gather_dma(t).wait()
            if t + 2 < NT:
                for d in idx_dmas(t + 2):
                    d.start()
            if t + 1 < NT:
                for d in idx_dmas(t + 1):
                    d.wait()
                remap(t + 1)
                gather_dma(t + 1).start()
            if len(pending_out) == 2:
                for d in pending_out.pop(0):
                    d.wait()
            compute(t)
            dmas = out_dmas(t)
            for d in dmas:
                d.start()
            pending_out.append(dmas)
        for dmas in pending_out:
            for d in dmas:
                d.wait()

    return k


def kernel(word_inputs, word_seq_lengths, word_embedding_table):
    idx5 = (word_inputs.astype(jnp.int32)
            .reshape(NW, BW, L, K)
            .transpose(2, 0, 3, 1)
            .reshape(L, NW, NIDX))
    tperm = _tc_relayout()(word_embedding_table.T)
    table_rows = tperm.reshape(VPAD, D)
    out = _sc_gather()(idx5, table_rows)
    return out.reshape(L, B, D).transpose(1, 0, 2)
